# baseline (speedup 1.0000x reference)
import jax
import jax.numpy as jnp
from jax.experimental import pallas as pl
from jax.experimental.pallas import tpu as pltpu

_BK = 512
_BR = 2048
_BM1 = 512


def _fused_kernel(adj_ref, seq_ref, o_ref, adjbf_ref, t_ref):
    s = pl.program_id(0)
    m = adjbf_ref.shape[0]
    np0 = (adjbf_ref.shape[1] // _BK) * (m // _BR)

    @pl.when(s < np0)
    def _():
        j = s // (m // _BR)
        r = s % (m // _BR)
        a = adj_ref[...].astype(jnp.bfloat16)
        adjbf_ref[pl.ds(r * _BR, _BR), pl.ds(j * _BK, _BK)] = a
        part = jnp.dot(a, seq_ref[...].astype(jnp.bfloat16),
                       preferred_element_type=jnp.float32)

        @pl.when(j == 0)
        def _():
            t_ref[pl.ds(r * _BR, _BR), :] = part

        @pl.when(j > 0)
        def _():
            t_ref[pl.ds(r * _BR, _BR), :] += part

    @pl.when(s >= np0)
    def _():
        i = s - np0
        a = adjbf_ref[pl.ds(i * _BM1, _BM1), :]
        o_ref[...] = jnp.dot(a, t_ref[...].astype(jnp.bfloat16),
                             preferred_element_type=jnp.float32)


def _fused_two_steps(adj, seq):
    m, k = adj.shape
    n = seq.shape[1]
    nk = k // _BK
    nr = m // _BR
    np0 = nk * nr
    return pl.pallas_call(
        _fused_kernel,
        grid=(np0 + m // _BM1,),
        in_specs=[
            pl.BlockSpec((_BR, _BK),
                         lambda s: (jnp.where(s < np0, s % nr, nr - 1),
                                    jnp.where(s < np0, s // nr, nk - 1))),
            pl.BlockSpec((_BK, n),
                         lambda s: (jnp.where(s < np0, s // nr, nk - 1), 0)),
        ],
        out_specs=pl.BlockSpec((_BM1, n),
                               lambda s: (jnp.where(s < np0, 0, s - np0), 0)),
        out_shape=jax.ShapeDtypeStruct((m, n), jnp.float32),
        scratch_shapes=[
            pltpu.VMEM((m, k), jnp.bfloat16),
            pltpu.VMEM((m, n), jnp.float32),
        ],
        compiler_params=pltpu.CompilerParams(
            vmem_limit_bytes=100 * 1024 * 1024),
    )(adj, seq)


def _mm_kernel(a_ref, b_ref, o_ref):
    o_ref[...] = jnp.dot(a_ref[...].astype(jnp.bfloat16),
                         b_ref[...].astype(jnp.bfloat16),
                         preferred_element_type=jnp.float32)


def _propagate(adj, s, bm=512):
    m, k = adj.shape
    n = s.shape[1]
    return pl.pallas_call(
        _mm_kernel,
        grid=(m // bm,),
        in_specs=[
            pl.BlockSpec((bm, k), lambda i: (i, 0)),
            pl.BlockSpec((k, n), lambda i: (0, 0)),
        ],
        out_specs=pl.BlockSpec((bm, n), lambda i: (i, 0)),
        out_shape=jax.ShapeDtypeStruct((m, n), jnp.float32),
    )(adj, s)


def kernel(seq, adj, propa_times):
    return jax.lax.cond(
        propa_times == 2,
        lambda: _fused_two_steps(adj, seq),
        lambda: jax.lax.fori_loop(
            0, propa_times, lambda _, s: _propagate(adj, s), seq),
    )


# speedup vs baseline: 1.0591x; 1.0591x over previous
import jax
import jax.numpy as jnp
from jax.experimental import pallas as pl
from jax.experimental.pallas import tpu as pltpu

_BM = 512


def _fused_kernel(adj_ref, seq_ref, o_ref, adjbf_ref, seqbf_ref, t_ref):
    s = pl.program_id(0)
    nblk = adjbf_ref.shape[0] // _BM

    @pl.when(s < nblk)
    def _():
        seqbf_ref[pl.ds(s * _BM, _BM), :] = seq_ref[...].astype(jnp.bfloat16)

    @pl.when(jnp.logical_and(s >= nblk, s < 2 * nblk))
    def _():
        i = s - nblk
        a = adj_ref[...].astype(jnp.bfloat16)
        adjbf_ref[pl.ds(i * _BM, _BM), :] = a
        t = jnp.dot(a, seqbf_ref[...],
                    preferred_element_type=jnp.float32)
        t_ref[pl.ds(i * _BM, _BM), :] = t.astype(jnp.bfloat16)

    @pl.when(s >= 2 * nblk)
    def _():
        i = s - 2 * nblk
        a = adjbf_ref[pl.ds(i * _BM, _BM), :]
        o_ref[...] = jnp.dot(a, t_ref[...],
                             preferred_element_type=jnp.float32)


def _fused_two_steps(adj, seq):
    m, k = adj.shape
    n = seq.shape[1]
    nblk = m // _BM
    return pl.pallas_call(
        _fused_kernel,
        grid=(3 * nblk,),
        in_specs=[
            pl.BlockSpec((_BM, k),
                         lambda s: (jnp.clip(s - nblk, 0, nblk - 1), 0)),
            pl.BlockSpec((_BM, n),
                         lambda s: (jnp.clip(s, 0, nblk - 1), 0)),
        ],
        out_specs=pl.BlockSpec((_BM, n),
                               lambda s: (jnp.clip(s - 2 * nblk, 0,
                                                   nblk - 1), 0)),
        out_shape=jax.ShapeDtypeStruct((m, n), jnp.float32),
        scratch_shapes=[
            pltpu.VMEM((m, k), jnp.bfloat16),
            pltpu.VMEM((m, n), jnp.bfloat16),
            pltpu.VMEM((m, n), jnp.bfloat16),
        ],
        compiler_params=pltpu.CompilerParams(
            vmem_limit_bytes=100 * 1024 * 1024),
    )(adj, seq)


def _mm_kernel(a_ref, b_ref, o_ref):
    o_ref[...] = jnp.dot(a_ref[...].astype(jnp.bfloat16),
                         b_ref[...].astype(jnp.bfloat16),
                         preferred_element_type=jnp.float32)


def _propagate(adj, s, bm=512):
    m, k = adj.shape
    n = s.shape[1]
    return pl.pallas_call(
        _mm_kernel,
        grid=(m // bm,),
        in_specs=[
            pl.BlockSpec((bm, k), lambda i: (i, 0)),
            pl.BlockSpec((k, n), lambda i: (0, 0)),
        ],
        out_specs=pl.BlockSpec((bm, n), lambda i: (i, 0)),
        out_shape=jax.ShapeDtypeStruct((m, n), jnp.float32),
    )(adj, s)


def kernel(seq, adj, propa_times):
    return jax.lax.cond(
        propa_times == 2,
        lambda: _fused_two_steps(adj, seq),
        lambda: jax.lax.fori_loop(
            0, propa_times, lambda _, s: _propagate(adj, s), seq),
    )


# speedup vs baseline: 1.1798x; 1.1140x over previous
import jax
import jax.numpy as jnp
from jax.experimental import pallas as pl
from jax.experimental.pallas import tpu as pltpu

_BM = 512


_BM1 = 1024


def _fused_kernel(adj_ref, seq_ref, o_ref, adjbf_ref, t_ref):
    s = pl.program_id(0)
    nblk = adjbf_ref.shape[0] // _BM

    @pl.when(s < nblk)
    def _():
        a = adj_ref[...].astype(jnp.bfloat16)
        adjbf_ref[pl.ds(s * _BM, _BM), :] = a
        t = jnp.dot(a, seq_ref[...],
                    preferred_element_type=jnp.float32)
        t_ref[pl.ds(s * _BM, _BM), :] = t.astype(jnp.bfloat16)

    @pl.when(s >= nblk)
    def _():
        i = s - nblk
        a = adjbf_ref[pl.ds(i * _BM1, _BM1), :]
        o_ref[...] = jnp.dot(a, t_ref[...],
                             preferred_element_type=jnp.float32)


def _fused_two_steps(adj, seq):
    m, k = adj.shape
    n = seq.shape[1]
    nblk = m // _BM
    seq = seq.astype(jnp.bfloat16)
    return pl.pallas_call(
        _fused_kernel,
        grid=(nblk + m // _BM1,),
        in_specs=[
            pl.BlockSpec((_BM, k),
                         lambda s: (jnp.clip(s, 0, nblk - 1), 0)),
            pl.BlockSpec((k, n), lambda s: (0, 0)),
        ],
        out_specs=pl.BlockSpec((_BM1, n),
                               lambda s: (jnp.clip(s - nblk, 0,
                                                   m // _BM1 - 1), 0)),
        out_shape=jax.ShapeDtypeStruct((m, n), jnp.float32),
        scratch_shapes=[
            pltpu.VMEM((m, k), jnp.bfloat16),
            pltpu.VMEM((m, n), jnp.bfloat16),
        ],
        compiler_params=pltpu.CompilerParams(
            vmem_limit_bytes=100 * 1024 * 1024),
    )(adj, seq)


def _mm_kernel(a_ref, b_ref, o_ref):
    o_ref[...] = jnp.dot(a_ref[...].astype(jnp.bfloat16),
                         b_ref[...].astype(jnp.bfloat16),
                         preferred_element_type=jnp.float32)


def _propagate(adj, s, bm=512):
    m, k = adj.shape
    n = s.shape[1]
    return pl.pallas_call(
        _mm_kernel,
        grid=(m // bm,),
        in_specs=[
            pl.BlockSpec((bm, k), lambda i: (i, 0)),
            pl.BlockSpec((k, n), lambda i: (0, 0)),
        ],
        out_specs=pl.BlockSpec((bm, n), lambda i: (i, 0)),
        out_shape=jax.ShapeDtypeStruct((m, n), jnp.float32),
    )(adj, s)


def kernel(seq, adj, propa_times):
    return jax.lax.cond(
        propa_times == 2,
        lambda: _fused_two_steps(adj, seq),
        lambda: jax.lax.fori_loop(
            0, propa_times, lambda _, s: _propagate(adj, s), seq),
    )
